# baseline (speedup 1.0000x reference)
import functools
import math

import jax
import jax.numpy as jnp
from jax import lax
from jax.experimental import pallas as pl
from jax.experimental.pallas import tpu as pltpu
from jax.experimental.pallas import tpu_sc as plsc

_M = 0.5
_COS_M = math.cos(_M)
_SIN_M = math.sin(_M)
_TH = math.cos(math.pi - _M)
_MM = math.sin(math.pi - _M) * _M
_EPS = 1e-07

_R = 1024
_C = 100000

_NC = 2
_NS = 16
_L = 16
_NW = _NC * _NS
_RPW = _R // _NW

_CSC = 99968
_W = 1408
_KC = _CSC // _W
_NJ = 4 * _KC
_NB = 4


def _phi_from_cos(c, sqrt_fn):
    c = jnp.minimum(jnp.maximum(c, -1.0 + _EPS), 1.0 - _EPS)
    sine = sqrt_fn(1.0 - c * c)
    phi = c * _COS_M - sine * _SIN_M
    return jnp.where(c > _TH, phi, c - _MM)


def _newton_sqrt(x):
    i = lax.bitcast_convert_type(x, jnp.int32)
    y = lax.bitcast_convert_type(
        lax.shift_right_arithmetic(i, 1) + 0x1FBD1DF5, jnp.float32)
    y = 0.5 * (y + x / y)
    y = 0.5 * (y + x / y)
    y = 0.5 * (y + x / y)
    return y


def _sc_body(cos_hbm, lbl_hbm, s_hbm, out_hbm,
             b0, b1, b2, b3, lblv, sv,
             i0, i1, i2, i3, o0, o1, o2, o3):
    bufs = (b0, b1, b2, b3)
    isems = (i0, i1, i2, i3)
    osems = (o0, o1, o2, o3)
    wid = lax.axis_index("s") * _NC + lax.axis_index("c")
    row0 = wid * _RPW
    pltpu.sync_copy(lbl_hbm.at[pl.ds(wid * 64, 64)], lblv)
    pltpu.sync_copy(s_hbm, sv)
    vs = sv[...]
    iota16 = lax.iota(jnp.int32, _L)
    rowsel = jnp.minimum(iota16, 7)

    def _src(j):
        r0 = row0 + (j // _KC) * 8
        c0 = (j % _KC) * _W
        return r0, c0

    def _start_in(j, b):
        r0, c0 = _src(j)
        pltpu.async_copy(cos_hbm.at[pl.ds(r0, 8), pl.ds(c0, _W)],
                         bufs[b], isems[b])

    def _wait_in(j, b):
        r0, c0 = _src(j)
        pltpu.make_async_copy(cos_hbm.at[pl.ds(r0, 8), pl.ds(c0, _W)],
                              bufs[b], isems[b]).wait()

    def _start_out(j, b):
        r0, c0 = _src(j)
        pltpu.async_copy(bufs[b],
                         out_hbm.at[pl.ds(r0, 8), pl.ds(c0, _W)], osems[b])

    def _wait_out(j, b):
        r0, c0 = _src(j)
        pltpu.make_async_copy(bufs[b],
                              out_hbm.at[pl.ds(r0, 8), pl.ds(c0, _W)],
                              osems[b]).wait()

    _start_in(0, 0)
    _start_in(1, 1)

    def _step(step, carry):
        for b in range(_NB):
            j = step * _NB + b
            buf = bufs[b]
            _wait_in(j, b)
            tr = j // _KC
            c0 = (j % _KC) * _W
            lbl16 = lblv[pl.ds(pl.multiple_of(tr * _L, _L), _L)]
            for k in range(8):
                crel = lbl16[k] - c0

                @pl.when((crel >= 0) & (crel < _W))
                def _():
                    cbase = pl.multiple_of((crel >> 4) << 4, _L)
                    lanev = jnp.zeros((_L,), jnp.int32) + (crel - cbase)
                    hit = iota16 == lanev
                    v = buf[k, pl.ds(cbase, _L)]
                    phi = _phi_from_cos(v, _newton_sqrt)
                    buf[k, pl.ds(cbase, _L)] = jnp.where(hit, phi, v)

            def _scale(i, _):
                for k in range(8):
                    sl = pl.ds(i * _L, _L)
                    buf[k, sl] = buf[k, sl] * vs
                return 0

            lax.fori_loop(0, _W // _L, _scale, 0)
            _start_out(j, b)

            bp = (b + 2) % _NB

            @pl.when((j >= 2) & (j + 2 < _NJ))
            def _():
                _wait_out(j - 2, bp)

            @pl.when(j + 2 < _NJ)
            def _():
                _start_in(j + 2, bp)
        return carry

    lax.fori_loop(0, _NJ // _NB, _step, 0)
    for b in range(_NB):
        _wait_out(_NJ - _NB + b, b)


@functools.cache
def _sc_kernel():
    mesh = plsc.VectorSubcoreMesh(core_axis_name="c", subcore_axis_name="s")
    return pl.kernel(
        _sc_body,
        out_type=jax.ShapeDtypeStruct((_R, _C), jnp.float32),
        mesh=mesh,
        scratch_types=(
            [pltpu.VMEM((8, _W), jnp.float32) for _ in range(_NB)]
            + [pltpu.VMEM((64,), jnp.int32), pltpu.VMEM((_L,), jnp.float32)]
            + [pltpu.SemaphoreType.DMA for _ in range(2 * _NB)]
        ),
    )


_BT = 128


def _tc_tail_body(s_ref, lbl_ref, x_ref, prev_ref, o_ref):
    del prev_ref
    cols = lax.broadcasted_iota(jnp.int32, (_R, _BT), 1) + _CSC
    mask = cols == lbl_ref[...]
    x = x_ref[...]
    phi = _phi_from_cos(x, jnp.sqrt)
    o_ref[...] = jnp.where(mask, phi, x) * s_ref[0]


def kernel(cosine, labels, s):
    lbl = labels.astype(jnp.int32)
    lbl_sp = jnp.reshape(
        jnp.pad(jnp.reshape(lbl, (128, 8)), ((0, 0), (0, 8))), (2048,))
    s_vec = jnp.full((_L,), s, jnp.float32)
    out_sc = _sc_kernel()(cosine, lbl_sp, s_vec)
    s_arr = jnp.reshape(jnp.asarray(s, jnp.float32), (1,))
    lbl2 = jnp.reshape(lbl, (_R, 1))
    return pl.pallas_call(
        _tc_tail_body,
        grid=(1,),
        in_specs=[
            pl.BlockSpec(memory_space=pltpu.SMEM),
            pl.BlockSpec((_R, 1), lambda i: (0, 0)),
            pl.BlockSpec((_R, _BT), lambda i: (0, _CSC // _BT)),
            pl.BlockSpec(memory_space=pltpu.MemorySpace.HBM),
        ],
        out_specs=pl.BlockSpec((_R, _BT), lambda i: (0, _CSC // _BT)),
        out_shape=jax.ShapeDtypeStruct((_R, _C), jnp.float32),
        input_output_aliases={3: 0},
    )(s_arr, lbl2, cosine, out_sc)
